# initial kernel scaffold (unmeasured)
import jax
import jax.numpy as jnp
from jax import lax
from jax.experimental import pallas as pl
from jax.experimental.pallas import tpu as pltpu

B, H, D = 8, 8, 64
NB = 64
BS = 16
LOCAL_PAGES = 64
T = LOCAL_PAGES * BS
NEG = -1e30


def kernel(Q, K, V, bt, lens):
    Qr = Q.reshape(B, H, D)
    Kr = K.reshape(T, H, D)
    Vr = V.reshape(T, H, D)
    lens2 = lens.reshape(B, 1)

    def body(q_ref, k_ref, v_ref, bt_ref, lens_ref, out_ref,
             send_ref, recv_ref, send_sem, recv_sem):
        my_x = lax.axis_index("x")
        my_y = lax.axis_index("y")
        my_z = lax.axis_index("z")
        partner = (my_x, 1 - my_y, my_z)

        barrier_sem = pltpu.get_barrier_semaphore()
        pl.semaphore_signal(barrier_sem, inc=1, device_id=partner,
                            device_id_type=pl.DeviceIdType.MESH)
        pl.semaphore_wait(barrier_sem, 1)

        bt_v = bt_ref[:]
        lens_v = lens_ref[:]
        j_ids = lax.broadcasted_iota(jnp.int32, (B, NB), 1)
        valid = j_ids < lens_v
        tok_page = (lax.broadcasted_iota(jnp.int32, (B, NB, T), 2) // BS
                    + my_y * LOCAL_PAGES)
        match = (bt_v[:, :, None] == tok_page) & valid[:, :, None]
        counts = jnp.sum(match.astype(jnp.float32), axis=1)

        q = q_ref[:]
        k = k_ref[:]
        v = v_ref[:]
        S = jnp.einsum("bhd,khd->bhk", q, k,
                       preferred_element_type=jnp.float32) * (D ** -0.5)
        has = counts[:, None, :] > 0.0
        S = jnp.where(has, S, NEG)
        m = jnp.max(S, axis=-1)
        p = jnp.exp(S - m[:, :, None]) * counts[:, None, :]
        s = jnp.sum(p, axis=-1)
        o = jnp.einsum("bhk,khd->bhd", p, v,
                       preferred_element_type=jnp.float32)

        m_b = jnp.broadcast_to(m[:, :, None], (B, H, D))
        s_b = jnp.broadcast_to(s[:, :, None], (B, H, D))
        send_ref[0, :, :, :] = o
        send_ref[1, :, :, :] = m_b
        send_ref[2, :, :, :] = s_b

        rdma = pltpu.make_async_remote_copy(
            src_ref=send_ref,
            dst_ref=recv_ref,
            send_sem=send_sem,
            recv_sem=recv_sem,
            device_id=partner,
            device_id_type=pl.DeviceIdType.MESH,
        )
        rdma.start()
        rdma.wait()

        o2 = recv_ref[0, :, :, :]
        m2 = recv_ref[1, :, :, :]
        s2 = recv_ref[2, :, :, :]
        m12 = jnp.maximum(m_b, m2)
        a1 = jnp.exp(m_b - m12)
        a2 = jnp.exp(m2 - m12)
        denom = s_b * a1 + s2 * a2
        res = (o * a1 + o2 * a2) / denom
        out_ref[:] = res[:, None, :, :]

    out_shape = jax.ShapeDtypeStruct((B, 1, H, D), jnp.float32)
    return pl.pallas_call(
        body,
        out_shape=out_shape,
        in_specs=[pl.BlockSpec(memory_space=pltpu.VMEM)] * 5,
        out_specs=pl.BlockSpec(memory_space=pltpu.VMEM),
        scratch_shapes=[
            pltpu.VMEM((3, B, H, D), jnp.float32),
            pltpu.VMEM((3, B, H, D), jnp.float32),
            pltpu.SemaphoreType.DMA,
            pltpu.SemaphoreType.DMA,
        ],
        compiler_params=pltpu.CompilerParams(collective_id=0),
    )(Qr, Kr, Vr, bt, lens2)


# baseline (device time: 15438 ns/iter reference)
import jax
import jax.numpy as jnp
from jax import lax
from jax.experimental import pallas as pl
from jax.experimental.pallas import tpu as pltpu

B, H, D = 8, 8, 64
NB = 64
BS = 16
LOCAL_PAGES = 64
T = LOCAL_PAGES * BS
NEG = -1e30
SCALE = D ** -0.5


def kernel(Q, K, V, bt, lens):
    Qh = Q.reshape(B, H, D).transpose(1, 0, 2)
    Kh = K.reshape(T, H, D).transpose(1, 0, 2)
    Vh = V.reshape(T, H, D).transpose(1, 0, 2)
    bt3 = bt.reshape(B, NB, 1)
    lens3 = lens.reshape(B, 1, 1)

    def body(q_ref, k_ref, v_ref, bt_ref, lens_ref, out_ref,
             send_ref, recv_ref, send_sem, recv_sem):
        my_x = lax.axis_index("x")
        my_y = lax.axis_index("y")
        my_z = lax.axis_index("z")
        partner = (my_x, 1 - my_y, my_z)

        barrier_sem = pltpu.get_barrier_semaphore()
        pl.semaphore_signal(barrier_sem, inc=1, device_id=partner,
                            device_id_type=pl.DeviceIdType.MESH)
        pl.semaphore_wait(barrier_sem, 1)

        bt_v = bt_ref[:]
        lens_v = lens_ref[:]
        j3 = lax.broadcasted_iota(jnp.int32, (B, NB, T), 1)
        tok_page = (lax.broadcasted_iota(jnp.int32, (B, NB, T), 2) // BS
                    + my_y * LOCAL_PAGES)
        match = (bt_v == tok_page) & (j3 < lens_v)
        counts = jnp.sum(match.astype(jnp.float32), axis=1)
        has = counts > 0.0

        part = []
        for h in range(H):
            q = q_ref[h]
            k = k_ref[h]
            v = v_ref[h]
            s_mat = lax.dot_general(
                q, k, (((1,), (1,)), ((), ())),
                preferred_element_type=jnp.float32) * SCALE
            s_mat = jnp.where(has, s_mat, NEG)
            m_h = jnp.max(s_mat, axis=-1, keepdims=True)
            p = jnp.exp(s_mat - m_h) * counts
            sum_h = jnp.sum(p, axis=-1, keepdims=True)
            o_h = lax.dot_general(
                p, v, (((1,), (0,)), ((), ())),
                preferred_element_type=jnp.float32)
            m_b = jnp.broadcast_to(m_h, (B, D))
            s_b = jnp.broadcast_to(sum_h, (B, D))
            send_ref[0, h, :, :] = o_h
            send_ref[1, h, :, :] = m_b
            send_ref[2, h, :, :] = s_b
            part.append((o_h, m_b, s_b))

        rdma = pltpu.make_async_remote_copy(
            src_ref=send_ref,
            dst_ref=recv_ref,
            send_sem=send_sem,
            recv_sem=recv_sem,
            device_id=partner,
            device_id_type=pl.DeviceIdType.MESH,
        )
        rdma.start()
        rdma.wait()

        for h in range(H):
            o1, m1, s1 = part[h]
            o2 = recv_ref[0, h, :, :]
            m2 = recv_ref[1, h, :, :]
            s2 = recv_ref[2, h, :, :]
            m12 = jnp.maximum(m1, m2)
            a1 = jnp.exp(m1 - m12)
            a2 = jnp.exp(m2 - m12)
            out_ref[h, :, :] = (o1 * a1 + o2 * a2) / (s1 * a1 + s2 * a2)

    out_shape = jax.ShapeDtypeStruct((H, B, D), jnp.float32)
    res = pl.pallas_call(
        body,
        out_shape=out_shape,
        in_specs=[pl.BlockSpec(memory_space=pltpu.VMEM)] * 5,
        out_specs=pl.BlockSpec(memory_space=pltpu.VMEM),
        scratch_shapes=[
            pltpu.VMEM((3, H, B, D), jnp.float32),
            pltpu.VMEM((3, H, B, D), jnp.float32),
            pltpu.SemaphoreType.DMA,
            pltpu.SemaphoreType.DMA,
        ],
        compiler_params=pltpu.CompilerParams(collective_id=0),
    )(Qh, Kh, Vh, bt3, lens3)
    return res.transpose(1, 0, 2).reshape(B, 1, H, D)


# device time: 15057 ns/iter; 1.0253x vs baseline; 1.0253x over previous
import jax
import jax.numpy as jnp
from jax import lax
from jax.experimental import pallas as pl
from jax.experimental.pallas import tpu as pltpu

B, H, D = 8, 8, 64
NB = 64
BS = 16
LOCAL_PAGES = 64
T = LOCAL_PAGES * BS
NEG = -1e30
SCALE = D ** -0.5


def kernel(Q, K, V, bt, lens):
    Qh = Q.reshape(B, H, D).transpose(1, 0, 2)
    Kh = K.reshape(T, H, D).transpose(1, 0, 2)
    Vh = V.reshape(T, H, D).transpose(1, 0, 2)
    bt3 = bt.reshape(B, NB, 1)
    lens3 = lens.reshape(B, 1, 1)

    def body(q_ref, k_ref, v_ref, bt_ref, lens_ref, out_ref,
             send_ref, recv_ref, send_sems, recv_sems):
        my_x = lax.axis_index("x")
        my_y = lax.axis_index("y")
        my_z = lax.axis_index("z")
        partner = (my_x, 1 - my_y, my_z)

        barrier_sem = pltpu.get_barrier_semaphore()
        pl.semaphore_signal(barrier_sem, inc=1, device_id=partner,
                            device_id_type=pl.DeviceIdType.MESH)

        bt_v = bt_ref[:]
        lens_v = lens_ref[:]
        j3 = lax.broadcasted_iota(jnp.int32, (B, NB, LOCAL_PAGES), 1)
        page3 = (lax.broadcasted_iota(jnp.int32, (B, NB, LOCAL_PAGES), 2)
                 + my_y * LOCAL_PAGES)
        match = (bt_v == page3) & (j3 < lens_v)
        counts_page = jnp.sum(match.astype(jnp.float32), axis=1)
        expand = (lax.broadcasted_iota(jnp.int32, (LOCAL_PAGES, T), 1) // BS
                  == lax.broadcasted_iota(jnp.int32, (LOCAL_PAGES, T), 0)
                  ).astype(jnp.float32)
        counts = lax.dot_general(
            counts_page, expand, (((1,), (0,)), ((), ())),
            preferred_element_type=jnp.float32)

        pl.semaphore_wait(barrier_sem, 1)

        part = []
        rdmas = []
        for h in range(H):
            q = q_ref[h]
            k = k_ref[h]
            v = v_ref[h]
            s_mat = lax.dot_general(
                q, k, (((1,), (1,)), ((), ())),
                preferred_element_type=jnp.float32) * SCALE
            m_h = jnp.max(s_mat, axis=-1, keepdims=True)
            p = jnp.exp(s_mat - m_h) * counts
            sum_h = jnp.sum(p, axis=-1, keepdims=True)
            o_h = lax.dot_general(
                p, v, (((1,), (0,)), ((), ())),
                preferred_element_type=jnp.float32)
            m_b = jnp.broadcast_to(m_h, (B, D))
            s_b = jnp.broadcast_to(sum_h, (B, D))
            send_ref[h, 0, :, :] = o_h
            send_ref[h, 1, :, :] = m_b
            send_ref[h, 2, :, :] = s_b
            part.append((o_h, m_b, s_b))
            rdma = pltpu.make_async_remote_copy(
                src_ref=send_ref.at[h],
                dst_ref=recv_ref.at[h],
                send_sem=send_sems.at[h],
                recv_sem=recv_sems.at[h],
                device_id=partner,
                device_id_type=pl.DeviceIdType.MESH,
            )
            rdma.start()
            rdmas.append(rdma)

        for h in range(H):
            rdmas[h].wait()
            o1, m1, s1 = part[h]
            o2 = recv_ref[h, 0, :, :]
            m2 = recv_ref[h, 1, :, :]
            s2 = recv_ref[h, 2, :, :]
            m12 = jnp.maximum(m1, m2)
            a1 = jnp.exp(m1 - m12)
            a2 = jnp.exp(m2 - m12)
            out_ref[h, :, :] = (o1 * a1 + o2 * a2) / (s1 * a1 + s2 * a2)

    out_shape = jax.ShapeDtypeStruct((H, B, D), jnp.float32)
    res = pl.pallas_call(
        body,
        out_shape=out_shape,
        in_specs=[pl.BlockSpec(memory_space=pltpu.VMEM)] * 5,
        out_specs=pl.BlockSpec(memory_space=pltpu.VMEM),
        scratch_shapes=[
            pltpu.VMEM((H, 3, B, D), jnp.float32),
            pltpu.VMEM((H, 3, B, D), jnp.float32),
            pltpu.SemaphoreType.DMA((H,)),
            pltpu.SemaphoreType.DMA((H,)),
        ],
        compiler_params=pltpu.CompilerParams(collective_id=0),
    )(Qh, Kh, Vh, bt3, lens3)
    return res.transpose(1, 0, 2).reshape(B, 1, H, D)


# device time: 14435 ns/iter; 1.0695x vs baseline; 1.0431x over previous
import jax
import jax.numpy as jnp
from jax import lax
from jax.experimental import pallas as pl
from jax.experimental.pallas import tpu as pltpu

B, H, D = 8, 8, 64
NB = 64
BS = 16
LOCAL_PAGES = 64
T = LOCAL_PAGES * BS
NEG = -1e30
SCALE = D ** -0.5
CHUNK = 4


def kernel(Q, K, V, bt, lens):
    Qh = Q.reshape(B, H, D).transpose(1, 0, 2)
    Kh = K.reshape(T, H, D).transpose(1, 0, 2)
    Vh = V.reshape(T, H, D).transpose(1, 0, 2)
    bt3 = bt.reshape(B, NB, 1)
    lens3 = lens.reshape(B, 1, 1)

    def body(q_ref, k_ref, v_ref, bt_ref, lens_ref, out_ref,
             send_ref, recv_ref, send_sems, recv_sems):
        my_x = lax.axis_index("x")
        my_y = lax.axis_index("y")
        my_z = lax.axis_index("z")
        partner = (my_x, 1 - my_y, my_z)

        barrier_sem = pltpu.get_barrier_semaphore()
        pl.semaphore_signal(barrier_sem, inc=1, device_id=partner,
                            device_id_type=pl.DeviceIdType.MESH)

        bt_v = bt_ref[:]
        lens_v = lens_ref[:]
        j3 = lax.broadcasted_iota(jnp.int32, (B, NB, LOCAL_PAGES), 1)
        page3 = (lax.broadcasted_iota(jnp.int32, (B, NB, LOCAL_PAGES), 2)
                 + my_y * LOCAL_PAGES)
        match = (bt_v == page3) & (j3 < lens_v)
        counts_page = jnp.sum(match.astype(jnp.float32), axis=1)
        expand = (lax.broadcasted_iota(jnp.int32, (LOCAL_PAGES, T), 1) // BS
                  == lax.broadcasted_iota(jnp.int32, (LOCAL_PAGES, T), 0)
                  ).astype(jnp.float32)
        counts = lax.dot_general(
            counts_page, expand, (((1,), (0,)), ((), ())),
            preferred_element_type=jnp.float32)

        part = []
        rdmas = []
        for h in range(H):
            q = q_ref[h]
            k = k_ref[h]
            v = v_ref[h]
            s_mat = lax.dot_general(
                q, k, (((1,), (1,)), ((), ())),
                preferred_element_type=jnp.float32) * SCALE
            m_h = jnp.max(s_mat, axis=-1, keepdims=True)
            p = jnp.exp(s_mat - m_h) * counts
            sum_h = jnp.sum(p, axis=-1, keepdims=True)
            o_h = lax.dot_general(
                p, v, (((1,), (0,)), ((), ())),
                preferred_element_type=jnp.float32)
            m_b = jnp.broadcast_to(m_h, (B, D))
            s_b = jnp.broadcast_to(sum_h, (B, D))
            send_ref[h, 0, :, :] = o_h
            send_ref[h, 1, :, :] = m_b
            send_ref[h, 2, :, :] = s_b
            part.append((o_h, m_b, s_b))
            if h == CHUNK - 1:
                pl.semaphore_wait(barrier_sem, 1)
            if h % CHUNK == CHUNK - 1:
                c = h // CHUNK
                rdma = pltpu.make_async_remote_copy(
                    src_ref=send_ref.at[pl.ds(c * CHUNK, CHUNK)],
                    dst_ref=recv_ref.at[pl.ds(c * CHUNK, CHUNK)],
                    send_sem=send_sems.at[c],
                    recv_sem=recv_sems.at[c],
                    device_id=partner,
                    device_id_type=pl.DeviceIdType.MESH,
                )
                rdma.start()
                rdmas.append(rdma)

        for h in range(H):
            if h % CHUNK == 0:
                rdmas[h // CHUNK].wait()
            o1, m1, s1 = part[h]
            o2 = recv_ref[h, 0, :, :]
            m2 = recv_ref[h, 1, :, :]
            s2 = recv_ref[h, 2, :, :]
            m12 = jnp.maximum(m1, m2)
            a1 = jnp.exp(m1 - m12)
            a2 = jnp.exp(m2 - m12)
            out_ref[h, :, :] = (o1 * a1 + o2 * a2) / (s1 * a1 + s2 * a2)

    out_shape = jax.ShapeDtypeStruct((H, B, D), jnp.float32)
    res = pl.pallas_call(
        body,
        out_shape=out_shape,
        in_specs=[pl.BlockSpec(memory_space=pltpu.VMEM)] * 5,
        out_specs=pl.BlockSpec(memory_space=pltpu.VMEM),
        scratch_shapes=[
            pltpu.VMEM((H, 3, B, D), jnp.float32),
            pltpu.VMEM((H, 3, B, D), jnp.float32),
            pltpu.SemaphoreType.DMA((H // CHUNK,)),
            pltpu.SemaphoreType.DMA((H // CHUNK,)),
        ],
        compiler_params=pltpu.CompilerParams(collective_id=0),
    )(Qh, Kh, Vh, bt3, lens3)
    return res.transpose(1, 0, 2).reshape(B, 1, H, D)
